# baseline (speedup 1.0000x reference)
import functools

import jax
import jax.numpy as jnp
from jax import lax
from jax.experimental import pallas as pl
from jax.experimental.pallas import tpu as pltpu
from jax.experimental.pallas import tpu_sc as plsc

NC = 2
NS = 16
NW = NC * NS
CH = 80
ZR = 32
SENT = 2147483647


def _mesh():
    return plsc.VectorSubcoreMesh(core_axis_name="c", subcore_axis_name="s")


def _zero_shared(acc, zbuf, sid, npad, d):
    zvec = jnp.zeros((16,), jnp.float32)

    def zrow(i, _):
        for j in range(d // 16):
            zbuf[i, pl.ds(16 * j, 16)] = zvec
        return 0

    lax.fori_loop(0, ZR, zrow, 0)
    rows_per_tile = npad // NS

    def zacc(i, _):
        pltpu.sync_copy(zbuf, acc.at[pl.ds(sid * rows_per_tile + i * ZR, ZR)])
        return 0

    lax.fori_loop(0, rows_per_tile // ZR, zacc, 0)


def _hop1_body(n_nodes, npad, d, ew, cap,
               ft_hbm, src_hbm, dst_hbm, dst2_hbm, pk_hbm,
               p_hbm, e3p_hbm, cnt_hbm,
               acc,
               src_c0, src_c1, dst_c0, dst_c1, dst2_c0, dst2_c1,
               pk_c0, pk_c1, sdst0, sdst1, sdst2_0, sdst2_1,
               rows0, rows1, e3p_v, zbuf, tmp_v,
               isem0, isem1, gsem0, gsem1, ssem0, ssem1):
    cid = lax.axis_index("c")
    sid = lax.axis_index("s")
    wid = cid * NS + sid
    dummy = n_nodes
    nch = ew // CH
    src_c = (src_c0, src_c1)
    dst_c = (dst_c0, dst_c1)
    dst2_c = (dst2_c0, dst2_c1)
    pk_c = (pk_c0, pk_c1)
    sdst = (sdst0, sdst1)
    sdst2 = (sdst2_0, sdst2_1)
    rows = (rows0, rows1)
    isem = (isem0, isem1)
    gsem = (gsem0, gsem1)
    ssem = (ssem0, ssem1)
    ebase = wid * ew

    def idx_copies(c, b):
        base = ebase + c * CH
        return (
            pltpu.make_async_copy(src_hbm.at[pl.ds(base, CH)], src_c[b], isem[b]),
            pltpu.make_async_copy(dst_hbm.at[pl.ds(base, CH)], dst_c[b], isem[b]),
            pltpu.make_async_copy(dst2_hbm.at[pl.ds(base, CH)], dst2_c[b], isem[b]),
            pltpu.make_async_copy(pk_hbm.at[pl.ds(base, CH)], pk_c[b], isem[b]),
        )

    def idx_issue(c, b):
        for cp in idx_copies(c, b):
            cp.start()

    def idx_wait(c, b):
        for cp in idx_copies(c, b):
            cp.wait()

    def gather_cp(b):
        return pltpu.make_async_copy(ft_hbm.at[src_c[b]], rows[b], gsem[b])

    def scatter_cps(b):
        return (pltpu.make_async_copy(rows[b], acc.at[sdst[b]], ssem[b]),
                pltpu.make_async_copy(rows[b], acc.at[sdst2[b]], ssem[b]))

    _zero_shared(acc, zbuf, sid, npad, d)
    plsc.subcore_barrier()

    idx_issue(0, 0)
    idx_issue(1, 1)
    idx_wait(0, 0)
    gather_cp(0).start()

    def iter_chunk(c, b, cnt):
        nb = 1 - b

        @pl.when(c > 0)
        def _():
            scatter_cps(nb)[1].wait()

        @pl.when(c + 1 < nch)
        def _():
            idx_wait(c + 1, nb)
            gather_cp(nb).start()

        for g in range(CH // 16):
            sl = pl.ds(g * 16, 16)
            sdst[b][sl] = dst_c[b][sl]
            sdst2[b][sl] = dst2_c[b][sl]

        gather_cp(b).wait()

        cp1, cp2 = scatter_cps(b)
        cp1.start(add=True)

        @pl.when(c + 2 < nch)
        def _():
            idx_issue(c + 2, b)

        for g in range(CH // 16):
            sl = pl.ds(g * 16, 16)
            v = pk_c[b][sl]
            sk, sv = plsc.sort_key_val(v, v)
            e3p_v[pl.ds(cnt, 16)] = sv
            cnt = cnt + plsc.all_reduce_population_count(sk != SENT)[0]

        cp1.wait()
        cp2.start(add=True)
        return cnt

    def pair(m, cnt):
        cnt = iter_chunk(2 * m, 0, cnt)
        cnt = iter_chunk(2 * m + 1, 1, cnt)
        return cnt

    cnt = lax.fori_loop(0, nch // 2, pair, jnp.int32(0))
    if nch % 2:
        cnt = iter_chunk(nch - 1, 0, cnt)
    scatter_cps((nch - 1) % 2)[1].wait()

    dvec = jnp.full((16,), dummy * 16384 + dummy, jnp.int32)
    for i in range(2 * CH // 16):
        e3p_v[pl.ds(cnt + i * 16, 16)] = dvec

    pltpu.sync_copy(e3p_v, e3p_hbm.at[pl.ds(wid * cap, cap)])
    tmp_v[...] = jnp.full((16,), cnt, jnp.int32)
    pltpu.sync_copy(tmp_v, cnt_hbm.at[pl.ds(wid * 16, 16)])

    plsc.subcore_barrier()
    rows_per_tile = npad // NS
    r0 = sid * rows_per_tile
    pltpu.sync_copy(acc.at[pl.ds(r0, rows_per_tile)],
                    p_hbm.at[pl.ds(cid * npad + r0, rows_per_tile)])


def _hop2_body(npad, d, cap,
               p_hbm, e3p_hbm, cnt_hbm,
               r_hbm, acc,
               pk_c0, pk_c1, srcA0, srcA1, srcB0, srcB1, dstv0, dstv1,
               rowsA0, rowsA1, rowsB0, rowsB1, zbuf, cnt_v,
               isem0, isem1, gsem0, gsem1, ssem0, ssem1):
    cid = lax.axis_index("c")
    sid = lax.axis_index("s")
    wid = cid * NS + sid
    pk_c = (pk_c0, pk_c1)
    srcA = (srcA0, srcA1)
    srcB = (srcB0, srcB1)
    dstv = (dstv0, dstv1)
    rowsA = (rowsA0, rowsA1)
    rowsB = (rowsB0, rowsB1)
    isem = (isem0, isem1)
    gsem = (gsem0, gsem1)
    ssem = (ssem0, ssem1)

    def idx_cp(c, b):
        return pltpu.make_async_copy(
            e3p_hbm.at[pl.ds(wid * cap + c * CH, CH)], pk_c[b], isem[b])

    def unpack(b):
        for g in range(CH // 16):
            sl = pl.ds(g * 16, 16)
            v = pk_c[b][sl]
            s = v >> 14
            srcA[b][sl] = s
            srcB[b][sl] = s + npad
            dstv[b][sl] = v & 16383

    def gather_cps(b):
        return (pltpu.make_async_copy(p_hbm.at[srcA[b]], rowsA[b], gsem[b]),
                pltpu.make_async_copy(p_hbm.at[srcB[b]], rowsB[b], gsem[b]))

    def scatter_cps(b):
        return (pltpu.make_async_copy(rowsA[b], acc.at[dstv[b]], ssem[b]),
                pltpu.make_async_copy(rowsB[b], acc.at[dstv[b]], ssem[b]))

    _zero_shared(acc, zbuf, sid, npad, d)
    plsc.subcore_barrier()

    pltpu.sync_copy(cnt_hbm.at[pl.ds(wid * 16, 16)], cnt_v)
    n = cnt_v[...][0]
    npairs = (n + 2 * CH - 1) // (2 * CH)
    nchk = 2 * npairs

    @pl.when(npairs > 0)
    def _():
        idx_cp(0, 0).start()
        idx_cp(1, 1).start()
        idx_cp(0, 0).wait()
        unpack(0)
        for cp in gather_cps(0):
            cp.start()

    def iter_chunk(c, b, not_first):
        nb = 1 - b

        def wait_prev():
            scatter_cps(nb)[1].wait()

        if not_first is None:
            wait_prev()
        else:
            pl.when(not_first)(wait_prev)

        @pl.when(c + 1 < nchk)
        def _():
            idx_cp(c + 1, nb).wait()
            unpack(nb)
            for cp in gather_cps(nb):
                cp.start()

        for cp in gather_cps(b):
            cp.wait()

        cpa, cpb = scatter_cps(b)
        cpa.start(add=True)

        @pl.when(c + 2 < nchk)
        def _():
            idx_cp(c + 2, b).start()

        cpa.wait()
        cpb.start(add=True)

    def pairbody(m, _):
        iter_chunk(2 * m, 0, m > 0)
        iter_chunk(2 * m + 1, 1, None)
        return 0

    lax.fori_loop(0, npairs, pairbody, 0)

    @pl.when(npairs > 0)
    def _():
        scatter_cps(1)[1].wait()

    plsc.subcore_barrier()
    rows_per_tile = npad // NS
    r0 = sid * rows_per_tile
    pltpu.sync_copy(acc.at[pl.ds(r0, rows_per_tile)],
                    r_hbm.at[pl.ds(cid * npad + r0, rows_per_tile)])


def _prep_tc(dummy, x_ref, w_ref, src_ref, dst_ref, ef_ref,
             ft_ref, dst2_ref, pk_ref):
    y = x_ref[...] * w_ref[...]
    ft_ref[...] = jnp.where(y > 0, y, jnp.exp(y) - 1.0)
    e = ef_ref[...]
    s = src_ref[...]
    t = dst_ref[...]
    m2 = (e == 0) | (e == 4) | (e == 5)
    dst2_ref[...] = jnp.where(m2, t, dummy)
    pk_ref[...] = jnp.where(e == 3, s * 16384 + t, SENT)


def _add_tc(a_ref, b_ref, o_ref):
    o_ref[...] = a_ref[...] + b_ref[...]


def kernel(graph_embedding, edge_index, e_feat, weight):
    n_nodes, d = graph_embedding.shape
    n_edges = e_feat.shape[0]
    assert n_edges % (NW * CH) == 0 and d % 16 == 0
    ew = n_edges // NW
    npad = ((n_nodes + 16 + NS * ZR - 1) // (NS * ZR)) * (NS * ZR)
    cap = ew + 2 * CH

    src = edge_index[0]
    dst = edge_index[1]

    ft, dst2, pk = pl.pallas_call(
        functools.partial(_prep_tc, n_nodes),
        out_shape=[
            jax.ShapeDtypeStruct((n_nodes, d), jnp.float32),
            jax.ShapeDtypeStruct((n_edges // 128, 128), jnp.int32),
            jax.ShapeDtypeStruct((n_edges // 128, 128), jnp.int32),
        ],
    )(graph_embedding, weight, src.reshape(-1, 128), dst.reshape(-1, 128),
      e_feat.reshape(-1, 128))

    hop1 = functools.partial(
        pl.kernel,
        out_type=[
            jax.ShapeDtypeStruct((NC * npad, d), jnp.float32),
            jax.ShapeDtypeStruct((NW * cap,), jnp.int32),
            jax.ShapeDtypeStruct((NW * 16,), jnp.int32),
        ],
        mesh=_mesh(),
        compiler_params=pltpu.CompilerParams(needs_layout_passes=False),
        scratch_types=(
            [pltpu.VMEM_SHARED((npad, d), jnp.float32)]
            + [pltpu.VMEM((CH,), jnp.int32)] * 12
            + [pltpu.VMEM((CH, d), jnp.float32)] * 2
            + [pltpu.VMEM((cap,), jnp.int32),
               pltpu.VMEM((ZR, d), jnp.float32),
               pltpu.VMEM((16,), jnp.int32)]
            + [pltpu.SemaphoreType.DMA] * 6
        ),
    )(functools.partial(_hop1_body, n_nodes, npad, d, ew, cap))
    p, e3p, cnt = hop1(ft, src, dst, dst2.reshape(-1), pk.reshape(-1))

    hop2 = functools.partial(
        pl.kernel,
        out_type=jax.ShapeDtypeStruct((NC * npad, d), jnp.float32),
        mesh=_mesh(),
        compiler_params=pltpu.CompilerParams(needs_layout_passes=False),
        scratch_types=(
            [pltpu.VMEM_SHARED((npad, d), jnp.float32)]
            + [pltpu.VMEM((CH,), jnp.int32)] * 8
            + [pltpu.VMEM((CH, d), jnp.float32)] * 4
            + [pltpu.VMEM((ZR, d), jnp.float32),
               pltpu.VMEM((16,), jnp.int32)]
            + [pltpu.SemaphoreType.DMA] * 6
        ),
    )(functools.partial(_hop2_body, npad, d, cap))
    r = hop2(p, e3p, cnt)

    res = pl.pallas_call(
        _add_tc,
        out_shape=jax.ShapeDtypeStruct((n_nodes, d), jnp.float32),
    )(r[:n_nodes], r[npad:npad + n_nodes])
    return res


# speedup vs baseline: 1.0065x; 1.0065x over previous
import functools

import jax
import jax.numpy as jnp
from jax import lax
from jax.experimental import pallas as pl
from jax.experimental.pallas import tpu as pltpu
from jax.experimental.pallas import tpu_sc as plsc

NC = 2
NS = 16
NW = NC * NS
CH = 80
ZR = 32
SENT = 2147483647


def _mesh():
    return plsc.VectorSubcoreMesh(core_axis_name="c", subcore_axis_name="s")


def _zero_shared(acc, zbuf, sid, npad, d):
    zvec = jnp.zeros((16,), jnp.float32)

    def zrow(i, _):
        for j in range(d // 16):
            zbuf[i, pl.ds(16 * j, 16)] = zvec
        return 0

    lax.fori_loop(0, ZR, zrow, 0)
    rows_per_tile = npad // NS

    def zacc(i, _):
        pltpu.sync_copy(zbuf, acc.at[pl.ds(sid * rows_per_tile + i * ZR, ZR)])
        return 0

    lax.fori_loop(0, rows_per_tile // ZR, zacc, 0)


def _hop1_body(n_nodes, npad, d, ew, cap,
               ft_hbm, src_hbm, dst_hbm, dst2_hbm, pk_hbm,
               p_hbm, e3p_hbm, cnt_hbm,
               acc,
               src_c0, src_c1, dst_c0, dst_c1, dst2_c0, dst2_c1,
               pk_c0, pk_c1, sdst0, sdst1, sdst2_0, sdst2_1,
               rows0, rows1, e3p_v, zbuf, tmp_v,
               isem0, isem1, gsem0, gsem1, ssem0, ssem1):
    cid = lax.axis_index("c")
    sid = lax.axis_index("s")
    wid = cid * NS + sid
    dummy = n_nodes
    nch = ew // CH
    src_c = (src_c0, src_c1)
    dst_c = (dst_c0, dst_c1)
    dst2_c = (dst2_c0, dst2_c1)
    pk_c = (pk_c0, pk_c1)
    sdst = (sdst0, sdst1)
    sdst2 = (sdst2_0, sdst2_1)
    rows = (rows0, rows1)
    isem = (isem0, isem1)
    gsem = (gsem0, gsem1)
    ssem = (ssem0, ssem1)
    ebase = wid * ew

    def idx_copies(c, b):
        base = ebase + c * CH
        return (
            pltpu.make_async_copy(src_hbm.at[pl.ds(base, CH)], src_c[b], isem[b]),
            pltpu.make_async_copy(dst_hbm.at[pl.ds(base, CH)], dst_c[b], isem[b]),
            pltpu.make_async_copy(dst2_hbm.at[pl.ds(base, CH)], dst2_c[b], isem[b]),
            pltpu.make_async_copy(pk_hbm.at[pl.ds(base, CH)], pk_c[b], isem[b]),
        )

    def idx_issue(c, b):
        for cp in idx_copies(c, b):
            cp.start()

    def idx_wait(c, b):
        for cp in idx_copies(c, b):
            cp.wait()

    def gather_cp(b):
        return pltpu.make_async_copy(ft_hbm.at[src_c[b]], rows[b], gsem[b])

    def scatter_cps(b):
        return (pltpu.make_async_copy(rows[b], acc.at[sdst[b]], ssem[b]),
                pltpu.make_async_copy(rows[b], acc.at[sdst2[b]], ssem[b]))

    _zero_shared(acc, zbuf, sid, npad, d)
    plsc.subcore_barrier()

    idx_issue(0, 0)
    idx_issue(1, 1)
    idx_wait(0, 0)
    gather_cp(0).start()

    def iter_chunk(c, b, cnt):
        nb = 1 - b

        @pl.when(c > 0)
        def _():
            for cp in scatter_cps(nb):
                cp.wait()

        @pl.when(c + 1 < nch)
        def _():
            idx_wait(c + 1, nb)
            gather_cp(nb).start()

        for g in range(CH // 16):
            sl = pl.ds(g * 16, 16)
            sdst[b][sl] = dst_c[b][sl]
            sdst2[b][sl] = dst2_c[b][sl]
            v = pk_c[b][sl]
            sk, sv = plsc.sort_key_val(v, v)
            e3p_v[pl.ds(cnt, 16)] = sv
            cnt = cnt + plsc.all_reduce_population_count(sk != SENT)[0]

        gather_cp(b).wait()

        @pl.when(c + 2 < nch)
        def _():
            idx_issue(c + 2, b)

        for cp in scatter_cps(b):
            cp.start(add=True)
        return cnt

    def pair(m, cnt):
        cnt = iter_chunk(2 * m, 0, cnt)
        cnt = iter_chunk(2 * m + 1, 1, cnt)
        return cnt

    cnt = lax.fori_loop(0, nch // 2, pair, jnp.int32(0))
    if nch % 2:
        cnt = iter_chunk(nch - 1, 0, cnt)
    for cp in scatter_cps((nch - 1) % 2):
        cp.wait()

    dvec = jnp.full((16,), dummy * 16384 + dummy, jnp.int32)
    for i in range(2 * CH // 16):
        e3p_v[pl.ds(cnt + i * 16, 16)] = dvec

    pltpu.sync_copy(e3p_v, e3p_hbm.at[pl.ds(wid * cap, cap)])
    tmp_v[...] = jnp.full((16,), cnt, jnp.int32)
    pltpu.sync_copy(tmp_v, cnt_hbm.at[pl.ds(wid * 16, 16)])

    plsc.subcore_barrier()
    rows_per_tile = npad // NS
    r0 = sid * rows_per_tile
    pltpu.sync_copy(acc.at[pl.ds(r0, rows_per_tile)],
                    p_hbm.at[pl.ds(cid * npad + r0, rows_per_tile)])


def _hop2_body(npad, d, cap,
               p_hbm, e3p_hbm, cnt_hbm,
               r_hbm, acc,
               pk_c0, pk_c1, srcA0, srcA1, srcB0, srcB1,
               dstv0, dstv1, dstw0, dstw1,
               rowsA0, rowsA1, rowsB0, rowsB1, zbuf, cnt_v,
               isem0, isem1, gsem0, gsem1, ssem0, ssem1):
    cid = lax.axis_index("c")
    sid = lax.axis_index("s")
    wid = cid * NS + sid
    pk_c = (pk_c0, pk_c1)
    srcA = (srcA0, srcA1)
    srcB = (srcB0, srcB1)
    dstv = (dstv0, dstv1)
    dstw = (dstw0, dstw1)
    rowsA = (rowsA0, rowsA1)
    rowsB = (rowsB0, rowsB1)
    isem = (isem0, isem1)
    gsem = (gsem0, gsem1)
    ssem = (ssem0, ssem1)

    def idx_cp(c, b):
        return pltpu.make_async_copy(
            e3p_hbm.at[pl.ds(wid * cap + c * CH, CH)], pk_c[b], isem[b])

    def unpack(b):
        for g in range(CH // 16):
            sl = pl.ds(g * 16, 16)
            rsl = pl.ds(((g + 3) % (CH // 16)) * 16, 16)
            v = pk_c[b][sl]
            s = v >> 14
            dt = v & 16383
            srcA[b][sl] = s
            dstv[b][sl] = dt
            srcB[b][rsl] = s + npad
            dstw[b][rsl] = dt

    def gather_cps(b):
        return (pltpu.make_async_copy(p_hbm.at[srcA[b]], rowsA[b], gsem[b]),
                pltpu.make_async_copy(p_hbm.at[srcB[b]], rowsB[b], gsem[b]))

    def scatter_cps(b):
        return (pltpu.make_async_copy(rowsA[b], acc.at[dstv[b]], ssem[b]),
                pltpu.make_async_copy(rowsB[b], acc.at[dstw[b]], ssem[b]))

    _zero_shared(acc, zbuf, sid, npad, d)
    plsc.subcore_barrier()

    pltpu.sync_copy(cnt_hbm.at[pl.ds(wid * 16, 16)], cnt_v)
    n = cnt_v[...][0]
    npairs = (n + 2 * CH - 1) // (2 * CH)
    nchk = 2 * npairs

    @pl.when(npairs > 0)
    def _():
        idx_cp(0, 0).start()
        idx_cp(1, 1).start()
        idx_cp(0, 0).wait()
        unpack(0)
        for cp in gather_cps(0):
            cp.start()

    def iter_chunk(c, b, not_first):
        nb = 1 - b

        def wait_prev():
            for cp in scatter_cps(nb):
                cp.wait()

        if not_first is None:
            wait_prev()
        else:
            pl.when(not_first)(wait_prev)

        @pl.when(c + 1 < nchk)
        def _():
            idx_cp(c + 1, nb).wait()
            unpack(nb)
            for cp in gather_cps(nb):
                cp.start()

        for cp in gather_cps(b):
            cp.wait()

        @pl.when(c + 2 < nchk)
        def _():
            idx_cp(c + 2, b).start()

        for cp in scatter_cps(b):
            cp.start(add=True)

    def pairbody(m, _):
        iter_chunk(2 * m, 0, m > 0)
        iter_chunk(2 * m + 1, 1, None)
        return 0

    lax.fori_loop(0, npairs, pairbody, 0)

    @pl.when(npairs > 0)
    def _():
        for cp in scatter_cps(1):
            cp.wait()

    plsc.subcore_barrier()
    rows_per_tile = npad // NS
    r0 = sid * rows_per_tile
    pltpu.sync_copy(acc.at[pl.ds(r0, rows_per_tile)],
                    r_hbm.at[pl.ds(cid * npad + r0, rows_per_tile)])


def _prep_tc(dummy, x_ref, w_ref, src_ref, dst_ref, ef_ref,
             ft_ref, dst2_ref, pk_ref):
    y = x_ref[...] * w_ref[...]
    ft_ref[...] = jnp.where(y > 0, y, jnp.exp(y) - 1.0)
    e = ef_ref[...]
    s = src_ref[...]
    t = dst_ref[...]
    m2 = (e == 0) | (e == 4) | (e == 5)
    dst2_ref[...] = jnp.where(m2, t, dummy)
    pk_ref[...] = jnp.where(e == 3, s * 16384 + t, SENT)


def _add_tc(a_ref, b_ref, o_ref):
    o_ref[...] = a_ref[...] + b_ref[...]


def kernel(graph_embedding, edge_index, e_feat, weight):
    n_nodes, d = graph_embedding.shape
    n_edges = e_feat.shape[0]
    assert n_edges % (NW * CH) == 0 and d % 16 == 0
    ew = n_edges // NW
    npad = ((n_nodes + 16 + NS * ZR - 1) // (NS * ZR)) * (NS * ZR)
    cap = ew + 2 * CH

    src = edge_index[0]
    dst = edge_index[1]

    ft, dst2, pk = pl.pallas_call(
        functools.partial(_prep_tc, n_nodes),
        out_shape=[
            jax.ShapeDtypeStruct((n_nodes, d), jnp.float32),
            jax.ShapeDtypeStruct((n_edges // 128, 128), jnp.int32),
            jax.ShapeDtypeStruct((n_edges // 128, 128), jnp.int32),
        ],
    )(graph_embedding, weight, src.reshape(-1, 128), dst.reshape(-1, 128),
      e_feat.reshape(-1, 128))

    hop1 = functools.partial(
        pl.kernel,
        out_type=[
            jax.ShapeDtypeStruct((NC * npad, d), jnp.float32),
            jax.ShapeDtypeStruct((NW * cap,), jnp.int32),
            jax.ShapeDtypeStruct((NW * 16,), jnp.int32),
        ],
        mesh=_mesh(),
        compiler_params=pltpu.CompilerParams(needs_layout_passes=False),
        scratch_types=(
            [pltpu.VMEM_SHARED((npad, d), jnp.float32)]
            + [pltpu.VMEM((CH,), jnp.int32)] * 12
            + [pltpu.VMEM((CH, d), jnp.float32)] * 2
            + [pltpu.VMEM((cap,), jnp.int32),
               pltpu.VMEM((ZR, d), jnp.float32),
               pltpu.VMEM((16,), jnp.int32)]
            + [pltpu.SemaphoreType.DMA] * 6
        ),
    )(functools.partial(_hop1_body, n_nodes, npad, d, ew, cap))
    p, e3p, cnt = hop1(ft, src, dst, dst2.reshape(-1), pk.reshape(-1))

    hop2 = functools.partial(
        pl.kernel,
        out_type=jax.ShapeDtypeStruct((NC * npad, d), jnp.float32),
        mesh=_mesh(),
        compiler_params=pltpu.CompilerParams(needs_layout_passes=False),
        scratch_types=(
            [pltpu.VMEM_SHARED((npad, d), jnp.float32)]
            + [pltpu.VMEM((CH,), jnp.int32)] * 10
            + [pltpu.VMEM((CH, d), jnp.float32)] * 4
            + [pltpu.VMEM((ZR, d), jnp.float32),
               pltpu.VMEM((16,), jnp.int32)]
            + [pltpu.SemaphoreType.DMA] * 6
        ),
    )(functools.partial(_hop2_body, npad, d, cap))
    r = hop2(p, e3p, cnt)

    res = pl.pallas_call(
        _add_tc,
        out_shape=jax.ShapeDtypeStruct((n_nodes, d), jnp.float32),
    )(r[:n_nodes], r[npad:npad + n_nodes])
    return res


# speedup vs baseline: 1.1893x; 1.1816x over previous
import functools

import jax
import jax.numpy as jnp
from jax import lax
from jax.experimental import pallas as pl
from jax.experimental.pallas import tpu as pltpu
from jax.experimental.pallas import tpu_sc as plsc

NC = 2
NS = 16
NW = NC * NS
CH = 80
ZR = 32
SENT = 2147483647


def _mesh():
    return plsc.VectorSubcoreMesh(core_axis_name="c", subcore_axis_name="s")


def _zero_shared(acc, zbuf, sem, sid, npad, d):
    zvec = jnp.zeros((16,), jnp.float32)

    def zrow(i, _):
        for j in range(d // 16):
            zbuf[i, pl.ds(16 * j, 16)] = zvec
        return 0

    lax.fori_loop(0, ZR, zrow, 0)
    rows_per_tile = npad // NS
    base = sid * rows_per_tile

    def zissue(i, _):
        pltpu.make_async_copy(zbuf, acc.at[pl.ds(base + i * ZR, ZR)], sem).start()
        return 0

    def zdrain(i, _):
        pltpu.make_async_copy(zbuf, acc.at[pl.ds(base, ZR)], sem).wait()
        return 0

    lax.fori_loop(0, rows_per_tile // ZR, zissue, 0)
    lax.fori_loop(0, rows_per_tile // ZR, zdrain, 0)


def _hop1_body(n_nodes, npad, d, ew, cap,
               ft_hbm, src_hbm, dst_hbm, dst2_hbm, pk_hbm,
               p_hbm, e3p_hbm, cnt_hbm,
               acc,
               src_c0, src_c1, dst_c0, dst_c1, dst2_c0, dst2_c1,
               pk_c0, pk_c1, sdst0, sdst1, sdst2_0, sdst2_1,
               rows0, rows1, e3p_v, zbuf, tmp_v,
               isem0, isem1, gsem0, gsem1, ssem0, ssem1):
    cid = lax.axis_index("c")
    sid = lax.axis_index("s")
    wid = cid * NS + sid
    dummy = n_nodes
    nch = ew // CH
    src_c = (src_c0, src_c1)
    dst_c = (dst_c0, dst_c1)
    dst2_c = (dst2_c0, dst2_c1)
    pk_c = (pk_c0, pk_c1)
    sdst = (sdst0, sdst1)
    sdst2 = (sdst2_0, sdst2_1)
    rows = (rows0, rows1)
    isem = (isem0, isem1)
    gsem = (gsem0, gsem1)
    ssem = (ssem0, ssem1)
    ebase = wid * ew

    def idx_copies(c, b):
        base = ebase + c * CH
        return (
            pltpu.make_async_copy(src_hbm.at[pl.ds(base, CH)], src_c[b], isem[b]),
            pltpu.make_async_copy(dst_hbm.at[pl.ds(base, CH)], dst_c[b], isem[b]),
            pltpu.make_async_copy(dst2_hbm.at[pl.ds(base, CH)], dst2_c[b], isem[b]),
            pltpu.make_async_copy(pk_hbm.at[pl.ds(base, CH)], pk_c[b], isem[b]),
        )

    def idx_issue(c, b):
        for cp in idx_copies(c, b):
            cp.start()

    def idx_wait(c, b):
        for cp in idx_copies(c, b):
            cp.wait()

    def gather_cps(b):
        return (pltpu.make_async_copy(ft_hbm.at[src_c[b]], rows[b], gsem[b]),)

    def scatter_cps(b):
        return (pltpu.make_async_copy(rows[b], acc.at[sdst[b]], ssem[b]),
                pltpu.make_async_copy(rows[b], acc.at[sdst2[b]], ssem[b]))

    with jax.named_scope("h1zero"):
        _zero_shared(acc, zbuf, gsem0, sid, npad, d)
        plsc.subcore_barrier()

    idx_issue(0, 0)
    idx_issue(1, 1)
    idx_wait(0, 0)
    for cp in gather_cps(0):
        cp.start()

    def iter_chunk(c, b, cnt):
        nb = 1 - b

        @pl.when(c > 0)
        def _():
            for cp in scatter_cps(nb):
                cp.wait()

        @pl.when(c + 1 < nch)
        def _():
            idx_wait(c + 1, nb)
            for cp in gather_cps(nb):
                cp.start()

        for g in range(CH // 16):
            sl = pl.ds(g * 16, 16)
            sdst[b][sl] = dst_c[b][sl]
            sdst2[b][sl] = dst2_c[b][sl]
            v = pk_c[b][sl]
            sk, sv = plsc.sort_key_val(v, v)
            e3p_v[pl.ds(cnt, 16)] = sv
            cnt = cnt + plsc.all_reduce_population_count(sk != SENT)[0]

        for cp in gather_cps(b):
            cp.wait()

        @pl.when(c + 2 < nch)
        def _():
            idx_issue(c + 2, b)

        for cp in scatter_cps(b):
            cp.start(add=True)
        return cnt

    def pair(m, cnt):
        cnt = iter_chunk(2 * m, 0, cnt)
        cnt = iter_chunk(2 * m + 1, 1, cnt)
        return cnt

    with jax.named_scope("h1loop"):
        cnt = lax.fori_loop(0, nch // 2, pair, jnp.int32(0))
        if nch % 2:
            cnt = iter_chunk(nch - 1, 0, cnt)
        for cp in scatter_cps((nch - 1) % 2):
            cp.wait()

    dvec = jnp.full((16,), dummy * 16384 + dummy, jnp.int32)
    for i in range(3 * CH // 16):
        e3p_v[pl.ds(cnt + i * 16, 16)] = dvec

    with jax.named_scope("h1wout"):
        pltpu.sync_copy(e3p_v, e3p_hbm.at[pl.ds(wid * cap, cap)])
        tmp_v[...] = jnp.full((16,), cnt, jnp.int32)
        pltpu.sync_copy(tmp_v, cnt_hbm.at[pl.ds(wid * 16, 16)])

        plsc.subcore_barrier()
        rows_per_tile = npad // NS
        r0 = sid * rows_per_tile
        pltpu.sync_copy(acc.at[pl.ds(r0, rows_per_tile)],
                        p_hbm.at[pl.ds(cid * npad + r0, rows_per_tile)])


def _hop2_body(npad, d, cap,
               twohop_hbm, e3p_hbm, cnt_hbm,
               r_hbm, acc,
               pk_c0, pk_c1, pk_c2, src0, src1, src2, dstv0, dstv1, dstv2,
               rows0, rows1, rows2, zbuf, cnt_v,
               isem0, isem1, isem2, gsem0, gsem1, gsem2,
               ssem0, ssem1, ssem2):
    cid = lax.axis_index("c")
    sid = lax.axis_index("s")
    wid = cid * NS + sid
    pk_c = (pk_c0, pk_c1, pk_c2)
    srcv = (src0, src1, src2)
    dstv = (dstv0, dstv1, dstv2)
    rows = (rows0, rows1, rows2)
    isem = (isem0, isem1, isem2)
    gsem = (gsem0, gsem1, gsem2)
    ssem = (ssem0, ssem1, ssem2)

    def idx_cp(c, b):
        return pltpu.make_async_copy(
            e3p_hbm.at[pl.ds(wid * cap + c * CH, CH)], pk_c[b], isem[b])

    def unpack(b):
        for g in range(CH // 16):
            sl = pl.ds(g * 16, 16)
            v = pk_c[b][sl]
            srcv[b][sl] = v >> 14
            dstv[b][sl] = v & 16383

    def gather_cp(b):
        return pltpu.make_async_copy(twohop_hbm.at[srcv[b]], rows[b], gsem[b])

    def scatter_cp(b):
        return pltpu.make_async_copy(rows[b], acc.at[dstv[b]], ssem[b])

    with jax.named_scope("h2zero"):
        _zero_shared(acc, zbuf, gsem0, sid, npad, d)
        plsc.subcore_barrier()

    pltpu.sync_copy(cnt_hbm.at[pl.ds(wid * 16, 16)], cnt_v)
    n = cnt_v[...][0]
    trips = (n + CH - 1) // CH

    def chunk(c, _):
        idx_cp(c, 0).start()
        idx_cp(c, 0).wait()
        unpack(0)
        gather_cp(0).start()
        gather_cp(0).wait()
        scatter_cp(0).start(add=True)
        scatter_cp(0).wait()
        return 0

    with jax.named_scope("h2loop"):
        lax.fori_loop(0, trips, chunk, 0)

    with jax.named_scope("h2wout"):
        plsc.subcore_barrier()
        rows_per_tile = npad // NS
        r0 = sid * rows_per_tile
        pltpu.sync_copy(acc.at[pl.ds(r0, rows_per_tile)],
                        r_hbm.at[pl.ds(cid * npad + r0, rows_per_tile)])


def _prep_tc(dummy, x_ref, w_ref, src_ref, dst_ref, ef_ref,
             ft_ref, dst2_ref, pk_ref):
    y = x_ref[...] * w_ref[...]
    ft_ref[...] = jnp.where(y > 0, y, jnp.exp(y) - 1.0)
    e = ef_ref[...]
    s = src_ref[...]
    t = dst_ref[...]
    m2 = (e == 0) | (e == 4) | (e == 5)
    dst2_ref[...] = jnp.where(m2, t, dummy)
    pk_ref[...] = jnp.where(e == 3, s * 16384 + t, SENT)


def _add_tc(a_ref, b_ref, o_ref):
    o_ref[...] = a_ref[...] + b_ref[...]


def kernel(graph_embedding, edge_index, e_feat, weight):
    n_nodes, d = graph_embedding.shape
    n_edges = e_feat.shape[0]
    assert n_edges % (NW * CH) == 0 and d % 16 == 0
    ew = n_edges // NW
    npad = ((n_nodes + 16 + NS * ZR - 1) // (NS * ZR)) * (NS * ZR)
    cap = ew + 3 * CH

    src = edge_index[0]
    dst = edge_index[1]

    ft, dst2, pk = pl.pallas_call(
        functools.partial(_prep_tc, n_nodes),
        out_shape=[
            jax.ShapeDtypeStruct((n_nodes, d), jnp.float32),
            jax.ShapeDtypeStruct((n_edges // 128, 128), jnp.int32),
            jax.ShapeDtypeStruct((n_edges // 128, 128), jnp.int32),
        ],
    )(graph_embedding, weight, src.reshape(-1, 128), dst.reshape(-1, 128),
      e_feat.reshape(-1, 128))

    hop1 = functools.partial(
        pl.kernel,
        out_type=[
            jax.ShapeDtypeStruct((NC * npad, d), jnp.float32),
            jax.ShapeDtypeStruct((NW * cap,), jnp.int32),
            jax.ShapeDtypeStruct((NW * 16,), jnp.int32),
        ],
        mesh=_mesh(),
        compiler_params=pltpu.CompilerParams(needs_layout_passes=False),
        scratch_types=(
            [pltpu.VMEM_SHARED((npad, d), jnp.float32)]
            + [pltpu.VMEM((CH,), jnp.int32)] * 12
            + [pltpu.VMEM((CH, d), jnp.float32)] * 2
            + [pltpu.VMEM((cap,), jnp.int32),
               pltpu.VMEM((ZR, d), jnp.float32),
               pltpu.VMEM((16,), jnp.int32)]
            + [pltpu.SemaphoreType.DMA] * 6
        ),
    )(functools.partial(_hop1_body, n_nodes, npad, d, ew, cap))
    p, e3p, cnt = hop1(ft, src, dst, dst2.reshape(-1), pk.reshape(-1))

    twohop = pl.pallas_call(
        _add_tc,
        out_shape=jax.ShapeDtypeStruct((npad, d), jnp.float32),
    )(p[:npad], p[npad:])

    hop2 = functools.partial(
        pl.kernel,
        out_type=jax.ShapeDtypeStruct((NC * npad, d), jnp.float32),
        mesh=_mesh(),
        compiler_params=pltpu.CompilerParams(needs_layout_passes=False),
        scratch_types=(
            [pltpu.VMEM_SHARED((npad, d), jnp.float32)]
            + [pltpu.VMEM((CH,), jnp.int32)] * 9
            + [pltpu.VMEM((CH, d), jnp.float32)] * 3
            + [pltpu.VMEM((ZR, d), jnp.float32),
               pltpu.VMEM((16,), jnp.int32)]
            + [pltpu.SemaphoreType.DMA] * 9
        ),
    )(functools.partial(_hop2_body, npad, d, cap))
    r = hop2(twohop, e3p, cnt)

    res = pl.pallas_call(
        _add_tc,
        out_shape=jax.ShapeDtypeStruct((n_nodes, d), jnp.float32),
    )(r[:n_nodes], r[npad:npad + n_nodes])
    return res
